# baseline (speedup 1.0000x reference)
import functools

import jax
import jax.numpy as jnp
from jax import lax
from jax.experimental import pallas as pl
from jax.experimental.pallas import tpu as pltpu
from jax.experimental.pallas import tpu_sc as plsc

HIDDEN = 4096
EXPERTS = 64
TOPK = 8
T_TOTAL = 4 * 8192
N_SPLIT = 4

TC_BLK = 256

NUM_CORES = 2
NUM_SUBCORES = 16
NW = NUM_CORES * NUM_SUBCORES
LANES = 16
SC_CHUNK = 512


def _tc_scores_body(h_ref, w_ref, out_ref):
    logits = lax.dot_general(
        h_ref[...], w_ref[...],
        dimension_numbers=(((1,), (1,)), ((), ())),
        preferred_element_type=jnp.float32,
    )
    out_ref[...] = jax.nn.sigmoid(logits)


def _tc_scores(h, weight, part, n_tok):
    blk0 = part * n_tok // TC_BLK
    return pl.pallas_call(
        _tc_scores_body,
        grid=(n_tok // TC_BLK,),
        in_specs=[
            pl.BlockSpec((TC_BLK, HIDDEN), lambda i: (i + blk0, 0)),
            pl.BlockSpec((EXPERTS, HIDDEN), lambda i: (0, 0)),
        ],
        out_specs=pl.BlockSpec((TC_BLK, EXPERTS), lambda i: (i, 0)),
        out_shape=jax.ShapeDtypeStruct((n_tok, EXPERTS), jnp.float32),
    )(h, weight)


def _merge_top16(ak, av, bk, bv):
    rk = lax.rev(bk, (0,))
    rv = lax.rev(bv, (0,))
    c = ak >= rk
    mk = jnp.where(c, ak, rk)
    mv = jnp.where(c, av, rv)
    return plsc.sort_key_val(mk, mv, descending=True)


def _sc_topk_body(tpw, chunk, scores_hbm, bias_hbm, idx_hbm, w_hbm,
                  scores_v, bias_v, idx_v, w_v):
    wid = lax.axis_index("s") * NUM_CORES + lax.axis_index("c")
    base = wid * tpw
    pltpu.sync_copy(bias_hbm, bias_v)

    iota = lax.iota(jnp.int32, LANES)
    mask8 = iota < TOPK
    col = jnp.where(mask8, iota, 0)
    bias_c = [bias_v[pl.ds(16 * i, 16)] for i in range(4)]
    idx_c = [iota + 16 * i for i in range(4)]

    def chunk_body(c, carry):
        cbase = base + c * chunk
        pltpu.sync_copy(scores_hbm.at[pl.ds(cbase, chunk)], scores_v)

        @plsc.parallel_loop(0, chunk, unroll=4)
        def _token_loop(t):
            srt = []
            for i in range(4):
                s = scores_v[t, pl.ds(16 * i, 16)]
                srt.append(plsc.sort_key_val(s + bias_c[i], idx_c[i],
                                             descending=True))
            k01, v01 = _merge_top16(*srt[0], *srt[1])
            k23, v23 = _merge_top16(*srt[2], *srt[3])
            _, fi = _merge_top16(k01, v01, k23, v23)
            t_s = jnp.full((LANES,), t, jnp.int32)
            w = plsc.load_gather(scores_v, [t_s, fi])
            wz = jnp.where(mask8, w, 0.0)
            wn = wz / (jnp.sum(wz) + 1e-20)
            plsc.store_scatter(idx_v, [t_s, col], fi, mask=mask8)
            plsc.store_scatter(w_v, [t_s, col], wn, mask=mask8)
        pltpu.sync_copy(idx_v, idx_hbm.at[pl.ds(cbase, chunk)])
        pltpu.sync_copy(w_v, w_hbm.at[pl.ds(cbase, chunk)])
        return carry

    lax.fori_loop(0, tpw // chunk, chunk_body, 0)


@functools.cache
def _make_sc_topk(n_tok):
    tpw = n_tok // NW
    chunk = min(SC_CHUNK, tpw)
    return functools.partial(
        pl.kernel,
        out_type=(
            jax.ShapeDtypeStruct((n_tok, TOPK), jnp.int32),
            jax.ShapeDtypeStruct((n_tok, TOPK), jnp.float32),
        ),
        mesh=plsc.VectorSubcoreMesh(core_axis_name="c", subcore_axis_name="s",
                                    num_cores=NUM_CORES,
                                    num_subcores=NUM_SUBCORES),
        scratch_types=[
            pltpu.VMEM((chunk, EXPERTS), jnp.float32),
            pltpu.VMEM((EXPERTS,), jnp.float32),
            pltpu.VMEM((chunk, TOPK), jnp.int32),
            pltpu.VMEM((chunk, TOPK), jnp.float32),
        ],
        compiler_params=pltpu.CompilerParams(needs_layout_passes=False,
                                             use_tc_tiling_on_sc=False),
    )(functools.partial(_sc_topk_body, tpw, chunk))


def kernel(hidden_states, weight, e_score_correction_bias):
    h = hidden_states.reshape(T_TOTAL, HIDDEN)
    t_half = T_TOTAL // N_SPLIT
    sc_call = _make_sc_topk(t_half)
    idx_parts, w_parts = [], []
    for p in range(N_SPLIT):
        scores = _tc_scores(h, weight, p, t_half)
        idx_p, w_p = sc_call(scores, e_score_correction_bias)
        idx_parts.append(idx_p)
        w_parts.append(w_p)
    topk_indices = jnp.concatenate(idx_parts, axis=0)
    topk_weights = jnp.concatenate(w_parts, axis=0)
    return topk_indices, topk_weights


# speedup vs baseline: 1.0610x; 1.0610x over previous
import functools

import jax
import jax.numpy as jnp
from jax import lax
from jax.experimental import pallas as pl
from jax.experimental.pallas import tpu as pltpu
from jax.experimental.pallas import tpu_sc as plsc

HIDDEN = 4096
EXPERTS = 64
TOPK = 8
T_TOTAL = 4 * 8192
PARTS = (12288, 10240, 6144, 3072, 1024)

TC_BLK = 512

NUM_CORES = 2
NUM_SUBCORES = 16
NW = NUM_CORES * NUM_SUBCORES
LANES = 16
SC_CHUNK = 512


def _tc_scores_body(h_ref, w_ref, out_ref):
    logits = lax.dot_general(
        h_ref[...], w_ref[...],
        dimension_numbers=(((1,), (1,)), ((), ())),
        preferred_element_type=jnp.float32,
    )
    out_ref[...] = jax.nn.sigmoid(logits)


def _tc_scores(h, weight, tok0, n_tok):
    blk0 = tok0 // TC_BLK
    return pl.pallas_call(
        _tc_scores_body,
        grid=(n_tok // TC_BLK,),
        in_specs=[
            pl.BlockSpec((TC_BLK, HIDDEN), lambda i: (i + blk0, 0)),
            pl.BlockSpec((EXPERTS, HIDDEN), lambda i: (0, 0)),
        ],
        out_specs=pl.BlockSpec((TC_BLK, EXPERTS), lambda i: (i, 0)),
        out_shape=jax.ShapeDtypeStruct((n_tok, EXPERTS), jnp.float32),
    )(h, weight)


def _merge_top16(ak, av, bk, bv):
    rk = lax.rev(bk, (0,))
    rv = lax.rev(bv, (0,))
    c = ak >= rk
    mk = jnp.where(c, ak, rk)
    mv = jnp.where(c, av, rv)
    return plsc.sort_key_val(mk, mv, descending=True)


def _sc_topk_body(tpw, chunk, scores_hbm, bias_hbm, idx_hbm, w_hbm,
                  scores_v, bias_v, idx_v, w_v):
    wid = lax.axis_index("s") * NUM_CORES + lax.axis_index("c")
    base = wid * tpw
    pltpu.sync_copy(bias_hbm, bias_v)

    iota = lax.iota(jnp.int32, LANES)
    mask8 = iota < TOPK
    col = jnp.where(mask8, iota, 0)
    bias_c = [bias_v[pl.ds(16 * i, 16)] for i in range(4)]
    idx_c = [iota + 16 * i for i in range(4)]

    def chunk_body(c, carry):
        cbase = base + c * chunk
        pltpu.sync_copy(scores_hbm.at[pl.ds(cbase, chunk)], scores_v)

        @plsc.parallel_loop(0, chunk, unroll=4)
        def _token_loop(t):
            srt = []
            for i in range(4):
                s = scores_v[t, pl.ds(16 * i, 16)]
                srt.append(plsc.sort_key_val(s + bias_c[i], idx_c[i],
                                             descending=True))
            k01, v01 = _merge_top16(*srt[0], *srt[1])
            k23, v23 = _merge_top16(*srt[2], *srt[3])
            _, fi = _merge_top16(k01, v01, k23, v23)
            t_s = jnp.full((LANES,), t, jnp.int32)
            w = plsc.load_gather(scores_v, [t_s, fi])
            wz = jnp.where(mask8, w, 0.0)
            wn = wz / (jnp.sum(wz) + 1e-20)
            plsc.store_scatter(idx_v, [t_s, col], fi, mask=mask8)
            plsc.store_scatter(w_v, [t_s, col], wn, mask=mask8)
        pltpu.sync_copy(idx_v, idx_hbm.at[pl.ds(cbase, chunk)])
        pltpu.sync_copy(w_v, w_hbm.at[pl.ds(cbase, chunk)])
        return carry

    lax.fori_loop(0, tpw // chunk, chunk_body, 0)


@functools.cache
def _make_sc_topk(n_tok):
    tpw = n_tok // NW
    chunk = min(SC_CHUNK, tpw)
    return functools.partial(
        pl.kernel,
        out_type=(
            jax.ShapeDtypeStruct((n_tok, TOPK), jnp.int32),
            jax.ShapeDtypeStruct((n_tok, TOPK), jnp.float32),
        ),
        mesh=plsc.VectorSubcoreMesh(core_axis_name="c", subcore_axis_name="s",
                                    num_cores=NUM_CORES,
                                    num_subcores=NUM_SUBCORES),
        scratch_types=[
            pltpu.VMEM((chunk, EXPERTS), jnp.float32),
            pltpu.VMEM((EXPERTS,), jnp.float32),
            pltpu.VMEM((chunk, TOPK), jnp.int32),
            pltpu.VMEM((chunk, TOPK), jnp.float32),
        ],
        compiler_params=pltpu.CompilerParams(needs_layout_passes=False,
                                             use_tc_tiling_on_sc=False),
    )(functools.partial(_sc_topk_body, tpw, chunk))


def kernel(hidden_states, weight, e_score_correction_bias):
    h = hidden_states.reshape(T_TOTAL, HIDDEN)
    idx_parts, w_parts = [], []
    tok0 = 0
    for n_tok in PARTS:
        scores = _tc_scores(h, weight, tok0, n_tok)
        idx_p, w_p = _make_sc_topk(n_tok)(scores, e_score_correction_bias)
        idx_parts.append(idx_p)
        w_parts.append(w_p)
        tok0 += n_tok
    topk_indices = jnp.concatenate(idx_parts, axis=0)
    topk_weights = jnp.concatenate(w_parts, axis=0)
    return topk_indices, topk_weights


# speedup vs baseline: 1.1039x; 1.0405x over previous
import functools

import jax
import jax.numpy as jnp
from jax import lax
from jax.experimental import pallas as pl
from jax.experimental.pallas import tpu as pltpu
from jax.experimental.pallas import tpu_sc as plsc

HIDDEN = 4096
EXPERTS = 64
TOPK = 8
T_TOTAL = 4 * 8192
PARTS = (12288, 8192, 8192, 4096)

TC_BLK = 512

NUM_CORES = 2
NUM_SUBCORES = 16
NW = NUM_CORES * NUM_SUBCORES
LANES = 16
SC_CHUNK = 512


def _tc_scores_body(h_ref, w_ref, out_ref):
    logits = lax.dot_general(
        h_ref[...], w_ref[...],
        dimension_numbers=(((1,), (1,)), ((), ())),
        preferred_element_type=jnp.float32,
    )
    out_ref[...] = jax.nn.sigmoid(logits)


def _tc_scores(h, weight, tok0, n_tok):
    blk0 = tok0 // TC_BLK
    return pl.pallas_call(
        _tc_scores_body,
        grid=(n_tok // TC_BLK,),
        in_specs=[
            pl.BlockSpec((TC_BLK, HIDDEN), lambda i: (i + blk0, 0)),
            pl.BlockSpec((EXPERTS, HIDDEN), lambda i: (0, 0)),
        ],
        out_specs=pl.BlockSpec((TC_BLK, EXPERTS), lambda i: (i, 0)),
        out_shape=jax.ShapeDtypeStruct((n_tok, EXPERTS), jnp.float32),
    )(h, weight)


def _merge_top16(ak, av, bk, bv):
    rk = lax.rev(bk, (0,))
    rv = lax.rev(bv, (0,))
    c = ak >= rk
    mk = jnp.where(c, ak, rk)
    mv = jnp.where(c, av, rv)
    return plsc.sort_key_val(mk, mv, descending=True)


def _sc_topk_body(tpw, chunk, scores_hbm, bias_hbm, idx_hbm, w_hbm,
                  scores_v, bias_v, idx_v, w_v):
    wid = lax.axis_index("s") * NUM_CORES + lax.axis_index("c")
    base = wid * tpw
    pltpu.sync_copy(bias_hbm, bias_v)

    iota = lax.iota(jnp.int32, LANES)
    mask8 = iota < TOPK
    col = jnp.where(mask8, iota, 0)
    bias_c = [bias_v[pl.ds(16 * i, 16)] for i in range(4)]
    idx_c = [iota + 16 * i for i in range(4)]

    def chunk_body(c, carry):
        cbase = base + c * chunk
        pltpu.sync_copy(scores_hbm.at[pl.ds(cbase, chunk)], scores_v)

        @plsc.parallel_loop(0, chunk, unroll=4)
        def _token_loop(t):
            srt = []
            for i in range(4):
                s = scores_v[t, pl.ds(16 * i, 16)]
                srt.append(plsc.sort_key_val(s + bias_c[i], idx_c[i],
                                             descending=True))
            k01, v01 = _merge_top16(*srt[0], *srt[1])
            k23, v23 = _merge_top16(*srt[2], *srt[3])
            _, fi = _merge_top16(k01, v01, k23, v23)
            t_s = jnp.full((LANES,), t, jnp.int32)
            w = plsc.load_gather(scores_v, [t_s, fi])
            wz = jnp.where(mask8, w, 0.0)
            wn = wz / (jnp.sum(wz) + 1e-20)
            plsc.store_scatter(idx_v, [t_s, col], fi, mask=mask8)
            plsc.store_scatter(w_v, [t_s, col], wn, mask=mask8)
        pltpu.sync_copy(idx_v, idx_hbm.at[pl.ds(cbase, chunk)])
        pltpu.sync_copy(w_v, w_hbm.at[pl.ds(cbase, chunk)])
        return carry

    lax.fori_loop(0, tpw // chunk, chunk_body, 0)


@functools.cache
def _make_sc_topk(n_tok):
    tpw = n_tok // NW
    chunk = min(SC_CHUNK, tpw)
    return functools.partial(
        pl.kernel,
        out_type=(
            jax.ShapeDtypeStruct((n_tok, TOPK), jnp.int32),
            jax.ShapeDtypeStruct((n_tok, TOPK), jnp.float32),
        ),
        mesh=plsc.VectorSubcoreMesh(core_axis_name="c", subcore_axis_name="s",
                                    num_cores=NUM_CORES,
                                    num_subcores=NUM_SUBCORES),
        scratch_types=[
            pltpu.VMEM((chunk, EXPERTS), jnp.float32),
            pltpu.VMEM((EXPERTS,), jnp.float32),
            pltpu.VMEM((chunk, TOPK), jnp.int32),
            pltpu.VMEM((chunk, TOPK), jnp.float32),
        ],
        compiler_params=pltpu.CompilerParams(needs_layout_passes=False,
                                             use_tc_tiling_on_sc=False),
    )(functools.partial(_sc_topk_body, tpw, chunk))


def kernel(hidden_states, weight, e_score_correction_bias):
    h = hidden_states.reshape(T_TOTAL, HIDDEN)
    idx_parts, w_parts = [], []
    tok0 = 0
    for n_tok in PARTS:
        scores = _tc_scores(h, weight, tok0, n_tok)
        idx_p, w_p = _make_sc_topk(n_tok)(scores, e_score_correction_bias)
        idx_parts.append(idx_p)
        w_parts.append(w_p)
        tok0 += n_tok
    topk_indices = jnp.concatenate(idx_parts, axis=0)
    topk_weights = jnp.concatenate(w_parts, axis=0)
    return topk_indices, topk_weights


# speedup vs baseline: 1.1288x; 1.0225x over previous
import functools

import jax
import jax.numpy as jnp
from jax import lax
from jax.experimental import pallas as pl
from jax.experimental.pallas import tpu as pltpu
from jax.experimental.pallas import tpu_sc as plsc

HIDDEN = 4096
EXPERTS = 64
TOPK = 8
T_TOTAL = 4 * 8192
PARTS = (8192, 8192, 8192, 8192)

TC_BLK = 512

NUM_CORES = 2
NUM_SUBCORES = 16
NW = NUM_CORES * NUM_SUBCORES
LANES = 16
SC_CHUNK = 512


def _tc_scores_body(h_ref, w_ref, out_ref):
    logits = lax.dot_general(
        h_ref[...], w_ref[...],
        dimension_numbers=(((1,), (1,)), ((), ())),
        preferred_element_type=jnp.float32,
    )
    out_ref[...] = jax.nn.sigmoid(logits)


def _tc_scores(h, weight, tok0, n_tok):
    blk0 = tok0 // TC_BLK
    return pl.pallas_call(
        _tc_scores_body,
        grid=(n_tok // TC_BLK,),
        in_specs=[
            pl.BlockSpec((TC_BLK, HIDDEN), lambda i: (i + blk0, 0)),
            pl.BlockSpec((EXPERTS, HIDDEN), lambda i: (0, 0)),
        ],
        out_specs=pl.BlockSpec((TC_BLK, EXPERTS), lambda i: (i, 0)),
        out_shape=jax.ShapeDtypeStruct((n_tok, EXPERTS), jnp.float32),
    )(h, weight)


def _merge_top16(ak, av, bk, bv):
    rk = lax.rev(bk, (0,))
    rv = lax.rev(bv, (0,))
    c = ak >= rk
    mk = jnp.where(c, ak, rk)
    mv = jnp.where(c, av, rv)
    return plsc.sort_key_val(mk, mv, descending=True)


def _sc_topk_body(tpw, chunk, scores_hbm, bias_hbm, idx_hbm, w_hbm,
                  scores_v, bias_v, idx_v, w_v):
    wid = lax.axis_index("s") * NUM_CORES + lax.axis_index("c")
    base = wid * tpw
    pltpu.sync_copy(bias_hbm, bias_v)

    iota = lax.iota(jnp.int32, LANES)
    mask8 = iota < TOPK
    col = jnp.where(mask8, iota, 0)
    bias_c = [bias_v[pl.ds(16 * i, 16)] for i in range(4)]
    idx_c = [iota + 16 * i for i in range(4)]

    def chunk_body(c, carry):
        cbase = base + c * chunk
        pltpu.sync_copy(scores_hbm.at[pl.ds(cbase, chunk)], scores_v)

        @plsc.parallel_loop(0, chunk, unroll=8)
        def _token_loop(t):
            srt = []
            for i in range(4):
                s = scores_v[t, pl.ds(16 * i, 16)]
                srt.append(plsc.sort_key_val(s + bias_c[i], idx_c[i],
                                             descending=True))
            k01, v01 = _merge_top16(*srt[0], *srt[1])
            k23, v23 = _merge_top16(*srt[2], *srt[3])
            _, fi = _merge_top16(k01, v01, k23, v23)
            t_s = jnp.full((LANES,), t, jnp.int32)
            w = plsc.load_gather(scores_v, [t_s, fi])
            wz = jnp.where(mask8, w, 0.0)
            wn = wz / (jnp.sum(wz) + 1e-20)
            plsc.store_scatter(idx_v, [t_s, col], fi, mask=mask8)
            plsc.store_scatter(w_v, [t_s, col], wn, mask=mask8)
        pltpu.sync_copy(idx_v, idx_hbm.at[pl.ds(cbase, chunk)])
        pltpu.sync_copy(w_v, w_hbm.at[pl.ds(cbase, chunk)])
        return carry

    lax.fori_loop(0, tpw // chunk, chunk_body, 0)


@functools.cache
def _make_sc_topk(n_tok):
    tpw = n_tok // NW
    chunk = min(SC_CHUNK, tpw)
    return functools.partial(
        pl.kernel,
        out_type=(
            jax.ShapeDtypeStruct((n_tok, TOPK), jnp.int32),
            jax.ShapeDtypeStruct((n_tok, TOPK), jnp.float32),
        ),
        mesh=plsc.VectorSubcoreMesh(core_axis_name="c", subcore_axis_name="s",
                                    num_cores=NUM_CORES,
                                    num_subcores=NUM_SUBCORES),
        scratch_types=[
            pltpu.VMEM((chunk, EXPERTS), jnp.float32),
            pltpu.VMEM((EXPERTS,), jnp.float32),
            pltpu.VMEM((chunk, TOPK), jnp.int32),
            pltpu.VMEM((chunk, TOPK), jnp.float32),
        ],
        compiler_params=pltpu.CompilerParams(needs_layout_passes=False,
                                             use_tc_tiling_on_sc=False),
    )(functools.partial(_sc_topk_body, tpw, chunk))


def kernel(hidden_states, weight, e_score_correction_bias):
    h = hidden_states.reshape(T_TOTAL, HIDDEN)
    idx_parts, w_parts = [], []
    tok0 = 0
    for n_tok in PARTS:
        scores = _tc_scores(h, weight, tok0, n_tok)
        idx_p, w_p = _make_sc_topk(n_tok)(scores, e_score_correction_bias)
        idx_parts.append(idx_p)
        w_parts.append(w_p)
        tok0 += n_tok
    topk_indices = jnp.concatenate(idx_parts, axis=0)
    topk_weights = jnp.concatenate(w_parts, axis=0)
    return topk_indices, topk_weights


# speedup vs baseline: 1.1318x; 1.0026x over previous
import functools

import jax
import jax.numpy as jnp
from jax import lax
from jax.experimental import pallas as pl
from jax.experimental.pallas import tpu as pltpu
from jax.experimental.pallas import tpu_sc as plsc

HIDDEN = 4096
EXPERTS = 64
TOPK = 8
T_TOTAL = 4 * 8192
PARTS = (8192, 8192, 8192, 8192)

TC_BLK = 512

NUM_CORES = 2
NUM_SUBCORES = 16
NW = NUM_CORES * NUM_SUBCORES
LANES = 16
SC_CHUNK = 512


def _tc_scores_body(h_ref, w_ref, out_ref):
    logits = lax.dot_general(
        h_ref[...], w_ref[...],
        dimension_numbers=(((1,), (1,)), ((), ())),
        preferred_element_type=jnp.float32,
    )
    out_ref[...] = jax.nn.sigmoid(logits)


def _tc_scores(h, weight, tok0, n_tok):
    blk0 = tok0 // TC_BLK
    return pl.pallas_call(
        _tc_scores_body,
        grid=(n_tok // TC_BLK,),
        in_specs=[
            pl.BlockSpec((TC_BLK, HIDDEN), lambda i: (i + blk0, 0)),
            pl.BlockSpec((EXPERTS, HIDDEN), lambda i: (0, 0)),
        ],
        out_specs=pl.BlockSpec((TC_BLK, EXPERTS), lambda i: (i, 0)),
        out_shape=jax.ShapeDtypeStruct((n_tok, EXPERTS), jnp.float32),
    )(h, weight)


def _rot8(x, rot8_idx):
    return jnp.take_along_axis(x, rot8_idx, axis=0)


def _merge_top8(ak, av, bk, bv, lane8, rot8_idx):
    ck = jnp.where(lane8, ak, lax.rev(bk, (0,)))
    cv = jnp.where(lane8, av, lax.rev(bv, (0,)))
    gk = _rot8(ck, rot8_idx)
    gv = _rot8(cv, rot8_idx)
    c = ck >= gk
    return jnp.where(c, ck, gk), jnp.where(c, cv, gv)


def _sc_topk_body(tpw, chunk, scores_hbm, bias_hbm, idx_hbm, w_hbm,
                  scores_v, bias_v, idx_v, w_v):
    wid = lax.axis_index("s") * NUM_CORES + lax.axis_index("c")
    base = wid * tpw
    pltpu.sync_copy(bias_hbm, bias_v)

    iota = lax.iota(jnp.int32, LANES)
    mask8 = iota < TOPK
    rot8_idx = (iota + 8) & 15
    col = jnp.where(mask8, iota, 0)
    bias_c = [bias_v[pl.ds(16 * i, 16)] for i in range(4)]
    idx_c = [iota + 16 * i for i in range(4)]

    def chunk_body(c, carry):
        cbase = base + c * chunk
        pltpu.sync_copy(scores_hbm.at[pl.ds(cbase, chunk)], scores_v)

        @plsc.parallel_loop(0, chunk, unroll=8)
        def _token_loop(t):
            srt = []
            for i in range(4):
                s = scores_v[t, pl.ds(16 * i, 16)]
                srt.append(plsc.sort_key_val(s + bias_c[i], idx_c[i],
                                             descending=True))
            k01, v01 = _merge_top8(*srt[0], *srt[1], mask8, rot8_idx)
            k23, v23 = _merge_top8(*srt[2], *srt[3], mask8, rot8_idx)
            fk = jnp.where(mask8, k01, _rot8(k23, rot8_idx))
            fv = jnp.where(mask8, v01, _rot8(v23, rot8_idx))
            _, fi = plsc.sort_key_val(fk, fv, descending=True)
            t_s = jnp.full((LANES,), t, jnp.int32)
            w = plsc.load_gather(scores_v, [t_s, fi])
            wz = jnp.where(mask8, w, 0.0)
            wn = wz / (jnp.sum(wz) + 1e-20)
            plsc.store_scatter(idx_v, [t_s, col], fi, mask=mask8)
            plsc.store_scatter(w_v, [t_s, col], wn, mask=mask8)
        pltpu.sync_copy(idx_v, idx_hbm.at[pl.ds(cbase, chunk)])
        pltpu.sync_copy(w_v, w_hbm.at[pl.ds(cbase, chunk)])
        return carry

    lax.fori_loop(0, tpw // chunk, chunk_body, 0)


@functools.cache
def _make_sc_topk(n_tok):
    tpw = n_tok // NW
    chunk = min(SC_CHUNK, tpw)
    return functools.partial(
        pl.kernel,
        out_type=(
            jax.ShapeDtypeStruct((n_tok, TOPK), jnp.int32),
            jax.ShapeDtypeStruct((n_tok, TOPK), jnp.float32),
        ),
        mesh=plsc.VectorSubcoreMesh(core_axis_name="c", subcore_axis_name="s",
                                    num_cores=NUM_CORES,
                                    num_subcores=NUM_SUBCORES),
        scratch_types=[
            pltpu.VMEM((chunk, EXPERTS), jnp.float32),
            pltpu.VMEM((EXPERTS,), jnp.float32),
            pltpu.VMEM((chunk, TOPK), jnp.int32),
            pltpu.VMEM((chunk, TOPK), jnp.float32),
        ],
        compiler_params=pltpu.CompilerParams(needs_layout_passes=False,
                                             use_tc_tiling_on_sc=False),
    )(functools.partial(_sc_topk_body, tpw, chunk))


def kernel(hidden_states, weight, e_score_correction_bias):
    h = hidden_states.reshape(T_TOTAL, HIDDEN)
    idx_parts, w_parts = [], []
    tok0 = 0
    for n_tok in PARTS:
        scores = _tc_scores(h, weight, tok0, n_tok)
        idx_p, w_p = _make_sc_topk(n_tok)(scores, e_score_correction_bias)
        idx_parts.append(idx_p)
        w_parts.append(w_p)
        tok0 += n_tok
    topk_indices = jnp.concatenate(idx_parts, axis=0)
    topk_weights = jnp.concatenate(w_parts, axis=0)
    return topk_indices, topk_weights
